# baseline (speedup 1.0000x reference)
import functools

import jax
import jax.numpy as jnp
from jax import lax
from jax.experimental import pallas as pl
from jax.experimental.pallas import tpu as pltpu
from jax.experimental.pallas import tpu_sc as plsc

B = 4
S = 8192
D = 768
EPS = 1e-12

NC = 2
NS = 16
NW = NC * NS
S_PER_W = S // NW
CH = 32
NCH = S_PER_W // CH
NG = CH // 16


def _ln_group(rows_v, pos_v, gb_v, g):
  ridx = lax.iota(jnp.int32, 16) + g * 16
  inv_d = jnp.float32(1.0 / D)

  def pass1(j, carry):
    vsum, vsq = carry
    cj = jnp.full((16,), j, jnp.int32)
    x = plsc.load_gather(rows_v, [ridx, cj]) + plsc.load_gather(pos_v, [ridx, cj])
    plsc.store_scatter(rows_v, [ridx, cj], x)
    return vsum + x, vsq + x * x

  zeros = jnp.zeros((16,), jnp.float32)
  vsum, vsq = lax.fori_loop(0, D, pass1, (zeros, zeros), unroll=8)

  mean = vsum * inv_d
  var = vsq * inv_d - mean * mean
  xv = var + jnp.float32(EPS)
  i = lax.bitcast_convert_type(xv, jnp.int32)
  i = jnp.int32(0x5F3759DF) - lax.shift_right_arithmetic(i, 1)
  y = lax.bitcast_convert_type(i, jnp.float32)
  for _ in range(3):
    y = y * (jnp.float32(1.5) - jnp.float32(0.5) * xv * y * y)
  rstd = y
  shift = mean * rstd

  def pass2(j, _):
    cj = jnp.full((16,), j, jnp.int32)
    x = plsc.load_gather(rows_v, [ridx, cj])
    gam = plsc.load_gather(gb_v, [cj])
    bet = plsc.load_gather(gb_v, [cj + D])
    o = (x * rstd - shift) * gam + bet
    plsc.store_scatter(rows_v, [ridx, cj], o)
    return 0

  lax.fori_loop(0, D, pass2, 0, unroll=8)


def _sc_kernel(ids_hbm, wemb_hbm, pemb_hbm, gb_hbm, out_hbm,
               idx_v, rows_v, pos_v, gb_v, sem):
  cid = lax.axis_index("c")
  sid = lax.axis_index("s")
  wid = sid * NC + cid
  s0 = wid * S_PER_W

  pltpu.sync_copy(gb_hbm, gb_v)

  def chunk_body(c, _):
    sb = s0 + c * CH
    pltpu.sync_copy(pemb_hbm.at[pl.ds(sb, CH)], pos_v)

    def b_body(b, _):
      base = b * S + sb
      pltpu.sync_copy(ids_hbm.at[pl.ds(base, CH)], idx_v)
      pltpu.async_copy(wemb_hbm.at[idx_v], rows_v, sem).wait()
      for g in range(NG):
        _ln_group(rows_v, pos_v, gb_v, g)
      pltpu.sync_copy(rows_v, out_hbm.at[pl.ds(base, CH)])
      return 0

    lax.fori_loop(0, B, b_body, 0)
    return 0

  lax.fori_loop(0, NCH, chunk_body, 0)


@jax.jit
def _run(ids_flat, word_emb, pos_emb, gb):
  mesh = plsc.VectorSubcoreMesh(
      core_axis_name="c", subcore_axis_name="s", num_cores=NC, num_subcores=NS)
  f = functools.partial(
      pl.kernel,
      out_type=jax.ShapeDtypeStruct((B * S, D), jnp.float32),
      mesh=mesh,
      scratch_types=[
          pltpu.VMEM((CH,), jnp.int32),
          pltpu.VMEM((CH, D), jnp.float32),
          pltpu.VMEM((CH, D), jnp.float32),
          pltpu.VMEM((2 * D,), jnp.float32),
          pltpu.SemaphoreType.DMA,
      ],
      compiler_params=pltpu.CompilerParams(needs_layout_passes=False),
  )(_sc_kernel)
  return f(ids_flat, word_emb, pos_emb, gb)


def kernel(input_ids, word_emb, pos_emb, ln_gamma, ln_beta):
  ids_flat = input_ids.reshape(B * S).astype(jnp.int32)
  gb = jnp.concatenate([ln_gamma, ln_beta]).astype(jnp.float32)
  out = _run(ids_flat, word_emb, pos_emb, gb)
  return out.reshape(B, S, D)


# speedup vs baseline: 11.4113x; 11.4113x over previous
import functools

import jax
import jax.numpy as jnp
from jax import lax
from jax.experimental import pallas as pl
from jax.experimental.pallas import tpu as pltpu
from jax.experimental.pallas import tpu_sc as plsc

B = 4
S = 8192
D = 768
EPS = 1e-12

NC = 2
NS = 16
NW = NC * NS
ROWS_PER_W = B * S // NW
CH = 64
NCH = ROWS_PER_W // CH

RB = 256
NSB = S // RB


def _sc_gather(ids_hbm, wemb_hbm, g_hbm, idx_v, b0, b1, si0, si1, so0, so1):
  cid = lax.axis_index("c")
  sid = lax.axis_index("s")
  wid = sid * NC + cid
  r0 = wid * ROWS_PER_W

  pltpu.sync_copy(ids_hbm.at[pl.ds(r0, ROWS_PER_W)], idx_v)

  bufs = (b0, b1)
  sin = (si0, si1)
  sout = (so0, so1)

  def start_in(c):
    return pltpu.async_copy(
        wemb_hbm.at[idx_v.at[pl.ds(c * CH, CH)]], bufs[c % 2], sin[c % 2])

  def start_out(c):
    return pltpu.async_copy(
        bufs[c % 2], g_hbm.at[pl.ds(r0 + c * CH, CH)], sout[c % 2])

  in_h = {0: start_in(0)}
  out_h = {}
  for c in range(NCH):
    in_h.pop(c).wait()
    if c >= 2:
      out_h.pop(c - 2).wait()
    if c + 1 < NCH:
      in_h[c + 1] = start_in(c + 1)
    out_h[c] = start_out(c)
  for c in out_h:
    out_h[c].wait()


@jax.jit
def _run(ids_flat, word_emb, pos_emb, gamma2d, beta2d):
  mesh = plsc.VectorSubcoreMesh(
      core_axis_name="c", subcore_axis_name="s", num_cores=NC, num_subcores=NS)
  gathered = functools.partial(
      pl.kernel,
      out_type=jax.ShapeDtypeStruct((B * S, D), jnp.float32),
      mesh=mesh,
      scratch_types=[
          pltpu.VMEM((ROWS_PER_W,), jnp.int32),
          pltpu.VMEM((CH, D), jnp.float32),
          pltpu.VMEM((CH, D), jnp.float32),
          pltpu.SemaphoreType.DMA,
          pltpu.SemaphoreType.DMA,
          pltpu.SemaphoreType.DMA,
          pltpu.SemaphoreType.DMA,
      ],
      compiler_params=pltpu.CompilerParams(needs_layout_passes=False),
  )(_sc_gather)(ids_flat, word_emb)

  def _ln_tc(rows_ref, pos_ref, gam_ref, bet_ref, out_ref):
    x = rows_ref[...] + pos_ref[...]
    mean = jnp.mean(x, axis=-1, keepdims=True)
    xc = x - mean
    var = jnp.mean(xc * xc, axis=-1, keepdims=True)
    out_ref[...] = xc * lax.rsqrt(var + EPS) * gam_ref[...] + bet_ref[...]

  out = pl.pallas_call(
      _ln_tc,
      grid=(NSB, B),
      in_specs=[
          pl.BlockSpec((RB, D), lambda i, j: (j * NSB + i, 0)),
          pl.BlockSpec((RB, D), lambda i, j: (i, 0)),
          pl.BlockSpec((1, D), lambda i, j: (0, 0)),
          pl.BlockSpec((1, D), lambda i, j: (0, 0)),
      ],
      out_specs=pl.BlockSpec((RB, D), lambda i, j: (j * NSB + i, 0)),
      out_shape=jax.ShapeDtypeStruct((B * S, D), jnp.float32),
  )(gathered, pos_emb, gamma2d, beta2d)
  return out


def kernel(input_ids, word_emb, pos_emb, ln_gamma, ln_beta):
  ids_flat = input_ids.reshape(B * S).astype(jnp.int32)
  out = _run(ids_flat, word_emb, pos_emb,
             ln_gamma.reshape(1, D).astype(jnp.float32),
             ln_beta.reshape(1, D).astype(jnp.float32))
  return out.reshape(B, S, D)
